# Initial kernel scaffold; baseline (speedup 1.0000x reference)
import jax
import jax.numpy as jnp
from jax.experimental import pallas as pl


def kernel(node_x, edge_x, node_batch, edge_batch, W1, b1, W2, b2):
    raise NotImplementedError("write your pallas kernel here")


# speedup vs baseline: 2.3092x; 2.3092x over previous
import functools

import jax
import jax.numpy as jnp
from jax import lax
from jax.experimental import pallas as pl
from jax.experimental.pallas import tpu as pltpu
from jax.experimental.pallas import tpu_sc as plsc

E = 3_200_000
N = 100_000
G = 64
H = 128
L = 32
NC, NS = 2, 16
NW = NC * NS
EPW = E // NW
CH = 4_000
NCH = EPW // CH
PAIRS = CH // 32
NPW = 3_200
NPAD = NW * NPW - N
GP = 80

_mesh = plsc.VectorSubcoreMesh(core_axis_name="c", subcore_axis_name="s")


@functools.partial(
    pl.kernel,
    out_type=(
        jax.ShapeDtypeStruct((NW, 10, 16 * G), jnp.float32),
        jax.ShapeDtypeStruct((NW, 16 * GP), jnp.float32),
    ),
    mesh=_mesh,
    scratch_types=(
        [pltpu.VMEM((2 * CH,), jnp.float32) for _ in range(2)]
        + [pltpu.VMEM((CH,), jnp.int32) for _ in range(2)]
        + [pltpu.VMEM((NPW,), jnp.int32)]
        + [pltpu.VMEM((16 * G,), jnp.float32) for _ in range(10)]
        + [pltpu.VMEM((16 * GP,), jnp.float32)]
        + [pltpu.SemaphoreType.DMA for _ in range(5)]
    ),
    compiler_params=pltpu.CompilerParams(needs_layout_passes=False),
)
def _sc_segment(eb_hbm, exf_hbm, nb_hbm, et_out, nt_out,
                exb0, exb1, ebb0, ebb1, nbb,
                ts0, tc0, tk0, tm0, te0, ts1, tc1, tk1, tm1, te1, accn,
                sx0, sx1, sb0, sb1, sn):
    wid = lax.axis_index("s") * NC + lax.axis_index("c")
    lane = lax.iota(jnp.int32, 16)
    iota2 = lane * 2
    lane_g = lane * G
    lane_gp = lane * GP
    zeros = jnp.zeros((16,), jnp.float32)
    ones = jnp.ones((16,), jnp.float32)

    tabs = (ts0, tc0, tk0, tm0, te0, ts1, tc1, tk1, tm1, te1)

    def zrow(r, carry):
        for t in tabs:
            t[pl.ds(r * 16, 16)] = zeros
        return carry
    lax.fori_loop(0, 16 * G // 16, zrow, 0)

    def zrow_n(r, carry):
        accn[pl.ds(r * 16, 16)] = zeros
        return carry
    lax.fori_loop(0, 16 * GP // 16, zrow_n, 0)

    ebase = wid * EPW
    exb = (exb0, exb1)
    ebb = (ebb0, ebb1)
    sx = (sx0, sx1)
    sb = (sb0, sb1)

    def start_chunk(j, slot):
        cpx = pltpu.async_copy(
            exf_hbm.at[pl.ds((ebase + j * CH) * 2, 2 * CH)], exb[slot], sx[slot])
        cpb = pltpu.async_copy(
            eb_hbm.at[pl.ds(ebase + j * CH, CH)], ebb[slot], sb[slot])
        return cpx, cpb

    pend = start_chunk(0, 0)

    pltpu.async_copy(nb_hbm.at[pl.ds(wid * NPW, NPW)], nbb, sn).wait()

    def nstep(i, carry):
        b = nbb[pl.ds(i * 16, 16)]
        plsc.addupdate_scatter(accn, [lane_gp + b], ones)
        return carry
    lax.fori_loop(0, NPW // 16, nstep, 0)

    for j in range(NCH):
        slot = j & 1
        if j + 1 < NCH:
            nxt = start_chunk(j + 1, 1 - slot)
        pend[0].wait()
        pend[1].wait()
        ex, eb = exb[slot], ebb[slot]

        def pair(i2, carry):
            for p, (ts_, tc_, tk_, tm_, te_) in enumerate(
                    ((ts0, tc0, tk0, tm0, te0), (ts1, tc1, tk1, tm1, te1))):
                s = i2 * 2 + p
                b = eb[pl.ds(s * 16, 16)]
                bi = lane_g + b
                ia = s * 32 + iota2
                a = plsc.load_gather(ex, [ia])
                o = plsc.load_gather(ex, [ia + 1])
                plsc.addupdate_scatter(ts_, [bi], a)
                plsc.addupdate_scatter(tc_, [bi], ones)
                kn = o >= 0.0
                plsc.addupdate_scatter(tk_, [bi], jnp.where(kn, ones, zeros))
                m = plsc.load_gather(tm_, [bi])
                plsc.store_scatter(tm_, [bi], jnp.maximum(m, a))
                ev = jnp.where(kn, o + 1.0, zeros)
                e_ = plsc.load_gather(te_, [bi])
                plsc.store_scatter(te_, [bi], jnp.maximum(e_, ev))
            return carry
        lax.fori_loop(0, PAIRS, pair, 0)
        if j + 1 < NCH:
            pend = nxt

    for k, t in enumerate(tabs):
        pltpu.sync_copy(t, et_out.at[wid, k])
    pltpu.sync_copy(accn, nt_out.at[wid])


def _finish_body(et_ref, nt_ref, w1t_ref, b1_ref, w2t_ref, b2_ref, out_ref):
    def rsum(x):
        return x.sum(axis=0).sum(axis=0)

    def rmax(x):
        return x.max(axis=0).max(axis=0)

    nn = rsum(nt_ref[:, :, :G])
    s_ = rsum(et_ref[:, 0] + et_ref[:, 5])
    c_ = rsum(et_ref[:, 1] + et_ref[:, 6])
    k_ = rsum(et_ref[:, 2] + et_ref[:, 7])
    m_ = rmax(jnp.maximum(et_ref[:, 3], et_ref[:, 8]))
    ev = rmax(jnp.maximum(et_ref[:, 4], et_ref[:, 9]))
    denom = jnp.maximum(c_, 1.0)
    feats_t = jnp.concatenate(
        [nn[None], c_[None], (s_ / denom)[None], m_[None],
         (k_ / denom)[None], ev[None]], axis=0)
    h_t = jnp.maximum(
        jnp.dot(w1t_ref[...], feats_t,
                preferred_element_type=jnp.float32) + b1_ref[...], 0.0)
    out = lax.dot_general(
        h_t, w2t_ref[...], (((0,), (1,)), ((), ())),
        preferred_element_type=jnp.float32)
    out_ref[...] = out + b2_ref[...]


def kernel(node_x, edge_x, node_batch, edge_batch, W1, b1, W2, b2):
    exf = edge_x.reshape(-1)
    nb = jnp.concatenate(
        [node_batch, jnp.full((NPAD,), G, jnp.int32)])
    et, nt = _sc_segment(edge_batch, exf, nb)
    et = et.reshape(NW, 10, 16, G)
    nt = nt.reshape(NW, 16, GP)
    return pl.pallas_call(
        _finish_body,
        out_shape=jax.ShapeDtypeStruct((G, L), jnp.float32),
    )(et, nt, W1.T, b1.reshape(H, 1), W2.T, b2.reshape(1, L))


# speedup vs baseline: 20.4037x; 8.8360x over previous
import functools

import jax
import jax.numpy as jnp
from jax import lax
from jax.experimental import pallas as pl
from jax.experimental.pallas import tpu as pltpu
from jax.experimental.pallas import tpu_sc as plsc

E = 3_200_000
N = 100_000
G = 64
H = 128
L = 32
NC, NS = 2, 16
NW = NC * NS
EPW = E // NW
CH = 4_000
NCH = EPW // CH
U = 5
NIT = CH // (16 * U)
NPW = 3_200
NPAD = NW * NPW - N
GP = 80
NT = 3 + 2 * U

_mesh = plsc.VectorSubcoreMesh(core_axis_name="c", subcore_axis_name="s")


@functools.partial(
    pl.kernel,
    out_type=(
        tuple(jax.ShapeDtypeStruct((NW, 16 * G), jnp.float32)
              for _ in range(NT))
        + (jax.ShapeDtypeStruct((NW, 16 * GP), jnp.float32),)
    ),
    mesh=_mesh,
    scratch_types=(
        [pltpu.VMEM((CH,), jnp.float32) for _ in range(4)]
        + [pltpu.VMEM((CH,), jnp.int32) for _ in range(2)]
        + [pltpu.VMEM((NPW,), jnp.int32)]
        + [pltpu.VMEM((16 * G,), jnp.float32) for _ in range(NT)]
        + [pltpu.VMEM((16 * GP,), jnp.float32)]
        + [pltpu.SemaphoreType.DMA for _ in range(7)]
    ),
    compiler_params=pltpu.CompilerParams(needs_layout_passes=False),
)
def _sc_segment(eb_hbm, exa_hbm, exo_hbm, nb_hbm,
                o_sum, o_cnt, o_knw, o_m0, o_m1, o_m2, o_m3, o_m4,
                o_e0, o_e1, o_e2, o_e3, o_e4, nt_out,
                exa0, exa1, exo0, exo1, ebb0, ebb1, nbb,
                tsum, tcnt, tknw,
                tm0, tm1, tm2, tm3, tm4,
                te0, te1, te2, te3, te4, accn,
                sa0, sa1, so0, so1, sb0, sb1, sn):
    wid = lax.axis_index("s") * NC + lax.axis_index("c")
    lane = lax.iota(jnp.int32, 16)
    lane_g = lane * G
    lane_gp = lane * GP
    zeros = jnp.zeros((16,), jnp.float32)
    ones = jnp.ones((16,), jnp.float32)

    tms = (tm0, tm1, tm2, tm3, tm4)
    tes = (te0, te1, te2, te3, te4)
    tabs = (tsum, tcnt, tknw) + tms + tes

    def zrow(r, carry):
        for t in tabs:
            t[pl.ds(r * 16, 16)] = zeros
        return carry
    lax.fori_loop(0, G, zrow, 0)

    def zrow_n(r, carry):
        accn[pl.ds(r * 16, 16)] = zeros
        return carry
    lax.fori_loop(0, GP, zrow_n, 0)

    ebase = wid * EPW
    exa = (exa0, exa1)
    exo = (exo0, exo1)
    ebb = (ebb0, ebb1)
    sa = (sa0, sa1)
    so = (so0, so1)
    sb = (sb0, sb1)

    def start_chunk(j, slot):
        sl = pl.ds(ebase + j * CH, CH)
        return (pltpu.async_copy(exa_hbm.at[sl], exa[slot], sa[slot]),
                pltpu.async_copy(exo_hbm.at[sl], exo[slot], so[slot]),
                pltpu.async_copy(eb_hbm.at[sl], ebb[slot], sb[slot]))

    pend = start_chunk(0, 0)

    pltpu.async_copy(nb_hbm.at[pl.ds(wid * NPW, NPW)], nbb, sn).wait()

    def nstep(i, carry):
        b = nbb[pl.ds(i * 16, 16)]
        plsc.addupdate_scatter(accn, [lane_gp + b], ones)
        return carry
    lax.fori_loop(0, NPW // 16, nstep, 0)

    for j in range(NCH):
        slot = j & 1
        if j + 1 < NCH:
            nxt = start_chunk(j + 1, 1 - slot)
        for cp in pend:
            cp.wait()
        xa, xo, eb = exa[slot], exo[slot], ebb[slot]

        def body(i, carry):
            for u in range(U):
                s = i * U + u
                sl = pl.ds(s * 16, 16)
                b = eb[sl]
                bi = lane_g + b
                a = xa[sl]
                o = xo[sl]
                plsc.addupdate_scatter(tsum, [bi], a)
                plsc.addupdate_scatter(tcnt, [bi], ones)
                kn = o >= 0.0
                plsc.addupdate_scatter(tknw, [bi], jnp.where(kn, ones, zeros))
                m = plsc.load_gather(tms[u], [bi])
                plsc.store_scatter(tms[u], [bi], jnp.maximum(m, a))
                ev = jnp.where(kn, o + 1.0, zeros)
                e_ = plsc.load_gather(tes[u], [bi])
                plsc.store_scatter(tes[u], [bi], jnp.maximum(e_, ev))
            return carry
        lax.fori_loop(0, NIT, body, 0)
        if j + 1 < NCH:
            pend = nxt

    outs = (o_sum, o_cnt, o_knw, o_m0, o_m1, o_m2, o_m3, o_m4,
            o_e0, o_e1, o_e2, o_e3, o_e4)
    for t, ot in zip(tabs, outs):
        pltpu.sync_copy(t, ot.at[wid])
    pltpu.sync_copy(accn, nt_out.at[wid])


def _finish_body(ts_ref, tc_ref, tk_ref, tm0, tm1, tm2, tm3, tm4,
                 te0, te1, te2, te3, te4, nt_ref,
                 w1t_ref, b1_ref, w2t_ref, b2_ref, out_ref):
    def rsum(x):
        return x.sum(axis=0).sum(axis=0)

    def rmax(x):
        return x.max(axis=0).max(axis=0)

    nn = rsum(nt_ref[:, :, :G])
    s_ = rsum(ts_ref[...])
    c_ = rsum(tc_ref[...])
    k_ = rsum(tk_ref[...])
    m_ = rmax(tm0[...])
    ev = rmax(te0[...])
    for tm_, te_ in ((tm1, te1), (tm2, te2), (tm3, te3), (tm4, te4)):
        m_ = jnp.maximum(m_, rmax(tm_[...]))
        ev = jnp.maximum(ev, rmax(te_[...]))
    denom = jnp.maximum(c_, 1.0)
    feats_t = jnp.concatenate(
        [nn[None], c_[None], (s_ / denom)[None], m_[None],
         (k_ / denom)[None], ev[None]], axis=0)
    h_t = jnp.maximum(
        jnp.dot(w1t_ref[...], feats_t,
                preferred_element_type=jnp.float32) + b1_ref[...], 0.0)
    out = lax.dot_general(
        h_t, w2t_ref[...], (((0,), (1,)), ((), ())),
        preferred_element_type=jnp.float32)
    out_ref[...] = out + b2_ref[...]


def kernel(node_x, edge_x, node_batch, edge_batch, W1, b1, W2, b2):
    arit = edge_x[:, 0]
    orig = edge_x[:, 1]
    nb = jnp.concatenate(
        [node_batch, jnp.full((NPAD,), G, jnp.int32)])
    *tables, nt = _sc_segment(edge_batch, arit, orig, nb)
    tables = [t.reshape(NW, 16, G) for t in tables]
    nt = nt.reshape(NW, 16, GP)
    return pl.pallas_call(
        _finish_body,
        out_shape=jax.ShapeDtypeStruct((G, L), jnp.float32),
    )(*tables, nt, W1.T, b1.reshape(H, 1), W2.T, b2.reshape(1, L))


# speedup vs baseline: 37.6664x; 1.8461x over previous
import functools

import jax
import jax.numpy as jnp
from jax import lax
from jax.experimental import pallas as pl
from jax.experimental.pallas import tpu as pltpu
from jax.experimental.pallas import tpu_sc as plsc

E = 3_200_000
N = 100_000
G = 64
H = 128
L = 32
NC, NS = 2, 16
NW = NC * NS
EPW = E // NW
CH = 10_000
NCH = EPW // CH
NPW = 3_200
NPAD = NW * NPW - N
GP = 80

_mesh = plsc.VectorSubcoreMesh(core_axis_name="c", subcore_axis_name="s")


@functools.partial(
    pl.kernel,
    out_type=(
        tuple(jax.ShapeDtypeStruct((NW, 16 * G), jnp.float32)
              for _ in range(5))
        + (jax.ShapeDtypeStruct((NW, 16 * GP), jnp.float32),)
    ),
    mesh=_mesh,
    scratch_types=(
        [pltpu.VMEM((CH,), jnp.float32) for _ in range(4)]
        + [pltpu.VMEM((CH,), jnp.int32) for _ in range(2)]
        + [pltpu.VMEM((NPW,), jnp.int32)]
        + [pltpu.VMEM((16 * G,), jnp.float32) for _ in range(5)]
        + [pltpu.VMEM((16 * GP,), jnp.float32)]
        + [pltpu.SemaphoreType.DMA for _ in range(7)]
    ),
    compiler_params=pltpu.CompilerParams(needs_layout_passes=False),
)
def _sc_segment(eb_hbm, exa_hbm, exo_hbm, nb_hbm,
                o_sum, o_cnt, o_knw, o_max, o_evt, nt_out,
                exa0, exa1, exo0, exo1, ebb0, ebb1, nbb,
                tsum, tcnt, tknw, tmax, tevt, accn,
                sa0, sa1, so0, so1, sb0, sb1, sn):
    wid = lax.axis_index("s") * NC + lax.axis_index("c")
    lane = lax.iota(jnp.int32, 16)
    lane_g = lane * G
    lane_gp = lane * GP
    zeros = jnp.zeros((16,), jnp.float32)
    ones = jnp.ones((16,), jnp.float32)

    tabs = (tsum, tcnt, tknw, tmax, tevt)

    def zrow(r, carry):
        for t in tabs:
            t[pl.ds(r * 16, 16)] = zeros
        return carry
    lax.fori_loop(0, G, zrow, 0)

    def zrow_n(r, carry):
        accn[pl.ds(r * 16, 16)] = zeros
        return carry
    lax.fori_loop(0, GP, zrow_n, 0)

    ebase = wid * EPW
    exa = (exa0, exa1)
    exo = (exo0, exo1)
    ebb = (ebb0, ebb1)
    sa = (sa0, sa1)
    so = (so0, so1)
    sb = (sb0, sb1)

    def start_chunk(j, slot):
        sl = pl.ds(ebase + j * CH, CH)
        return (pltpu.async_copy(exa_hbm.at[sl], exa[slot], sa[slot]),
                pltpu.async_copy(exo_hbm.at[sl], exo[slot], so[slot]),
                pltpu.async_copy(eb_hbm.at[sl], ebb[slot], sb[slot]))

    pend = start_chunk(0, 0)

    pltpu.async_copy(nb_hbm.at[pl.ds(wid * NPW, NPW)], nbb, sn).wait()

    def nstep(i, carry):
        b = nbb[pl.ds(i * 16, 16)]
        plsc.addupdate_scatter(accn, [lane_gp + b], ones)
        return carry
    lax.fori_loop(0, NPW // 16, nstep, 0)

    def flush(bprev, rs, rc, rk, rm, re):
        bi = lane_g + bprev
        plsc.addupdate_scatter(tsum, [bi], rs)
        plsc.addupdate_scatter(tcnt, [bi], rc)
        plsc.addupdate_scatter(tknw, [bi], rk)
        m = plsc.load_gather(tmax, [bi])
        plsc.store_scatter(tmax, [bi], jnp.maximum(m, rm))
        e_ = plsc.load_gather(tevt, [bi])
        plsc.store_scatter(tevt, [bi], jnp.maximum(e_, re))

    carry0 = (jnp.zeros((16,), jnp.int32), zeros, zeros, zeros, zeros, zeros)

    carry = carry0
    for j in range(NCH):
        slot = j & 1
        if j + 1 < NCH:
            nxt = start_chunk(j + 1, 1 - slot)
        for cp in pend:
            cp.wait()
        xa, xo, eb = exa[slot], exo[slot], ebb[slot]

        def step(i, c):
            bprev, rs, rc, rk, rm, re = c
            sl = pl.ds(i * 16, 16)
            b = eb[sl]
            a = xa[sl]
            o = xo[sl]
            same = jnp.all(b == bprev)

            @pl.when(jnp.logical_not(same))
            def _():
                flush(bprev, rs, rc, rk, rm, re)

            keep = jnp.where(same, ones, zeros)
            rs = rs * keep + a
            rc = rc * keep + ones
            kn = o >= 0.0
            rk = rk * keep + jnp.where(kn, ones, zeros)
            rm = jnp.maximum(rm * keep, a)
            re = jnp.maximum(re * keep, jnp.where(kn, o + 1.0, zeros))
            return (b, rs, rc, rk, rm, re)

        carry = lax.fori_loop(0, CH // 16, step, carry)
        if j + 1 < NCH:
            pend = nxt

    flush(*carry)

    outs = (o_sum, o_cnt, o_knw, o_max, o_evt)
    for t, ot in zip(tabs, outs):
        pltpu.sync_copy(t, ot.at[wid])
    pltpu.sync_copy(accn, nt_out.at[wid])


def _finish_body(ts_ref, tc_ref, tk_ref, tm_ref, te_ref, nt_ref,
                 w1t_ref, b1_ref, w2t_ref, b2_ref, out_ref):
    def rsum(x):
        return x.sum(axis=0).sum(axis=0)

    def rmax(x):
        return x.max(axis=0).max(axis=0)

    nn = rsum(nt_ref[:, :, :G])
    s_ = rsum(ts_ref[...])
    c_ = rsum(tc_ref[...])
    k_ = rsum(tk_ref[...])
    m_ = rmax(tm_ref[...])
    ev = rmax(te_ref[...])
    denom = jnp.maximum(c_, 1.0)
    feats_t = jnp.concatenate(
        [nn[None], c_[None], (s_ / denom)[None], m_[None],
         (k_ / denom)[None], ev[None]], axis=0)
    h_t = jnp.maximum(
        jnp.dot(w1t_ref[...], feats_t,
                preferred_element_type=jnp.float32) + b1_ref[...], 0.0)
    out = lax.dot_general(
        h_t, w2t_ref[...], (((0,), (1,)), ((), ())),
        preferred_element_type=jnp.float32)
    out_ref[...] = out + b2_ref[...]


def kernel(node_x, edge_x, node_batch, edge_batch, W1, b1, W2, b2):
    arit = edge_x[:, :1].reshape(E)
    orig = edge_x[:, 1:].reshape(E)
    nb = jnp.concatenate(
        [node_batch, jnp.full((NPAD,), G, jnp.int32)])
    *tables, nt = _sc_segment(edge_batch, arit, orig, nb)
    tables = [t.reshape(NW, 16, G) for t in tables]
    nt = nt.reshape(NW, 16, GP)
    return pl.pallas_call(
        _finish_body,
        out_shape=jax.ShapeDtypeStruct((G, L), jnp.float32),
    )(*tables, nt, W1.T, b1.reshape(H, 1), W2.T, b2.reshape(1, L))


# speedup vs baseline: 112.6352x; 2.9903x over previous
import functools

import jax
import jax.numpy as jnp
from jax import lax
from jax.experimental import pallas as pl
from jax.experimental.pallas import tpu as pltpu
from jax.experimental.pallas import tpu_sc as plsc

E = 3_200_000
N = 100_000
G = 64
H = 128
L = 32
NC, NS = 2, 16
NW = NC * NS
EPW = E // NW
CH = 10_000
NCH = EPW // CH
K = 25
NPW = 3_200
NPAD = NW * NPW - N
GP = 80

_mesh = plsc.VectorSubcoreMesh(core_axis_name="c", subcore_axis_name="s")


@functools.partial(
    pl.kernel,
    out_type=(
        tuple(jax.ShapeDtypeStruct((NW, 16 * G), jnp.float32)
              for _ in range(5))
        + (jax.ShapeDtypeStruct((NW, 16 * GP), jnp.float32),)
    ),
    mesh=_mesh,
    scratch_types=(
        [pltpu.VMEM((CH,), jnp.float32) for _ in range(4)]
        + [pltpu.VMEM((CH,), jnp.int32) for _ in range(2)]
        + [pltpu.VMEM((NPW,), jnp.int32)]
        + [pltpu.VMEM((16 * G,), jnp.float32) for _ in range(5)]
        + [pltpu.VMEM((16 * GP,), jnp.float32)]
        + [pltpu.SemaphoreType.DMA for _ in range(7)]
    ),
    compiler_params=pltpu.CompilerParams(needs_layout_passes=False),
)
def _sc_segment(eb_hbm, exa_hbm, exo_hbm, nb_hbm,
                o_sum, o_cnt, o_knw, o_max, o_evt, nt_out,
                exa0, exa1, exo0, exo1, ebb0, ebb1, nbb,
                tsum, tcnt, tknw, tmax, tevt, accn,
                sa0, sa1, so0, so1, sb0, sb1, sn):
    wid = lax.axis_index("s") * NC + lax.axis_index("c")
    lane = lax.iota(jnp.int32, 16)
    lane_g = lane * G
    lane_gp = lane * GP
    zeros = jnp.zeros((16,), jnp.float32)
    ones = jnp.ones((16,), jnp.float32)

    tabs = (tsum, tcnt, tknw, tmax, tevt)

    def zrow(r, carry):
        for t in tabs:
            t[pl.ds(r * 16, 16)] = zeros
        return carry
    lax.fori_loop(0, G, zrow, 0)

    def zrow_n(r, carry):
        accn[pl.ds(r * 16, 16)] = zeros
        return carry
    lax.fori_loop(0, GP, zrow_n, 0)

    ebase = wid * EPW
    exa = (exa0, exa1)
    exo = (exo0, exo1)
    ebb = (ebb0, ebb1)
    sa = (sa0, sa1)
    so = (so0, so1)
    sb = (sb0, sb1)

    def start_chunk(j, slot):
        sl = pl.ds(ebase + j * CH, CH)
        return (pltpu.async_copy(exa_hbm.at[sl], exa[slot], sa[slot]),
                pltpu.async_copy(exo_hbm.at[sl], exo[slot], so[slot]),
                pltpu.async_copy(eb_hbm.at[sl], ebb[slot], sb[slot]))

    pend = start_chunk(0, 0)

    pltpu.async_copy(nb_hbm.at[pl.ds(wid * NPW, NPW)], nbb, sn).wait()

    def nstep(i, carry):
        b = nbb[pl.ds(i * 16, 16)]
        plsc.addupdate_scatter(accn, [lane_gp + b], ones)
        return carry
    lax.fori_loop(0, NPW // 16, nstep, 0)

    def flush(bprev, rs, rc, rk, rm, re):
        bi = lane_g + bprev
        plsc.addupdate_scatter(tsum, [bi], rs)
        plsc.addupdate_scatter(tcnt, [bi], rc)
        plsc.addupdate_scatter(tknw, [bi], rk)
        m = plsc.load_gather(tmax, [bi])
        plsc.store_scatter(tmax, [bi], jnp.maximum(m, rm))
        e_ = plsc.load_gather(tevt, [bi])
        plsc.store_scatter(tevt, [bi], jnp.maximum(e_, re))

    kf = jnp.float32(K)
    carry0 = (jnp.zeros((16,), jnp.int32),) + (zeros,) * 10

    carry = carry0
    for j in range(NCH):
        slot = j & 1
        if j + 1 < NCH:
            nxt = start_chunk(j + 1, 1 - slot)
        for cp in pend:
            cp.wait()
        xa, xo, eb = exa[slot], exo[slot], ebb[slot]

        def block(i, c):
            bprev0 = c[0]
            b_first = eb[pl.ds(i * (16 * K), 16)]
            b_last = eb[pl.ds(i * (16 * K) + 16 * (K - 1), 16)]
            uni = jnp.logical_and(jnp.all(b_first == bprev0),
                                  jnp.all(b_last == bprev0))

            def fast(c):
                bp, rs0, rs1, rc0, rc1, rk0, rk1, rm0, rm1, re0, re1 = c
                r = [rs0, rs1, rk0, rk1, rm0, rm1, re0, re1]
                for u in range(K):
                    sl = pl.ds((i * K + u) * 16, 16)
                    a = xa[sl]
                    o = xo[sl]
                    kn = o >= 0.0
                    p = u & 1
                    r[0 + p] = r[0 + p] + a
                    r[2 + p] = r[2 + p] + jnp.where(kn, ones, zeros)
                    r[4 + p] = jnp.maximum(r[4 + p], a)
                    r[6 + p] = jnp.maximum(r[6 + p],
                                           jnp.where(kn, o + 1.0, zeros))
                return (bp, r[0], r[1], rc0 + kf, rc1,
                        r[2], r[3], r[4], r[5], r[6], r[7])

            def slow(c):
                bprev, rs0, rs1, rc0, rc1, rk0, rk1, rm0, rm1, re0, re1 = c
                rs, rc = rs0 + rs1, rc0 + rc1
                rk = rk0 + rk1
                rm, re = jnp.maximum(rm0, rm1), jnp.maximum(re0, re1)
                for u in range(K):
                    sl = pl.ds((i * K + u) * 16, 16)
                    b = eb[sl]
                    a = xa[sl]
                    o = xo[sl]
                    chg = b != bprev
                    bi = lane_g + bprev
                    plsc.addupdate_scatter(tsum, [bi], rs, mask=chg)
                    plsc.addupdate_scatter(tcnt, [bi], rc, mask=chg)
                    plsc.addupdate_scatter(tknw, [bi], rk, mask=chg)
                    m = plsc.load_gather(tmax, [bi])
                    plsc.store_scatter(tmax, [bi], jnp.maximum(m, rm),
                                       mask=chg)
                    e_ = plsc.load_gather(tevt, [bi])
                    plsc.store_scatter(tevt, [bi], jnp.maximum(e_, re),
                                       mask=chg)
                    gone = jnp.where(chg, zeros, ones)
                    kn = o >= 0.0
                    rs = rs * gone + a
                    rc = rc * gone + ones
                    rk = rk * gone + jnp.where(kn, ones, zeros)
                    rm = jnp.maximum(rm * gone, a)
                    re = jnp.maximum(re * gone,
                                     jnp.where(kn, o + 1.0, zeros))
                    bprev = b
                return (bprev, rs, zeros, rc, zeros,
                        rk, zeros, rm, zeros, re, zeros)

            return lax.cond(uni, fast, slow, c)

        carry = lax.fori_loop(0, CH // (16 * K), block, carry)
        if j + 1 < NCH:
            pend = nxt

    bp, rs0, rs1, rc0, rc1, rk0, rk1, rm0, rm1, re0, re1 = carry
    flush(bp, rs0 + rs1, rc0 + rc1, rk0 + rk1,
          jnp.maximum(rm0, rm1), jnp.maximum(re0, re1))

    outs = (o_sum, o_cnt, o_knw, o_max, o_evt)
    for t, ot in zip(tabs, outs):
        pltpu.sync_copy(t, ot.at[wid])
    pltpu.sync_copy(accn, nt_out.at[wid])


def _finish_body(ts_ref, tc_ref, tk_ref, tm_ref, te_ref, nt_ref,
                 w1t_ref, b1_ref, w2t_ref, b2_ref, out_ref):
    def rsum(x):
        return x.sum(axis=0).sum(axis=0)

    def rmax(x):
        return x.max(axis=0).max(axis=0)

    nn = rsum(nt_ref[:, :, :G])
    s_ = rsum(ts_ref[...])
    c_ = rsum(tc_ref[...])
    k_ = rsum(tk_ref[...])
    m_ = rmax(tm_ref[...])
    ev = rmax(te_ref[...])
    denom = jnp.maximum(c_, 1.0)
    feats_t = jnp.concatenate(
        [nn[None], c_[None], (s_ / denom)[None], m_[None],
         (k_ / denom)[None], ev[None]], axis=0)
    h_t = jnp.maximum(
        jnp.dot(w1t_ref[...], feats_t,
                preferred_element_type=jnp.float32) + b1_ref[...], 0.0)
    out = lax.dot_general(
        h_t, w2t_ref[...], (((0,), (1,)), ((), ())),
        preferred_element_type=jnp.float32)
    out_ref[...] = out + b2_ref[...]


def kernel(node_x, edge_x, node_batch, edge_batch, W1, b1, W2, b2):
    arit = edge_x[:, :1].reshape(E)
    orig = edge_x[:, 1:].reshape(E)
    nb = jnp.concatenate(
        [node_batch, jnp.full((NPAD,), G, jnp.int32)])
    *tables, nt = _sc_segment(edge_batch, arit, orig, nb)
    tables = [t.reshape(NW, 16, G) for t in tables]
    nt = nt.reshape(NW, 16, GP)
    return pl.pallas_call(
        _finish_body,
        out_shape=jax.ShapeDtypeStruct((G, L), jnp.float32),
    )(*tables, nt, W1.T, b1.reshape(H, 1), W2.T, b2.reshape(1, L))


# speedup vs baseline: 126.1682x; 1.1201x over previous
import functools

import jax
import jax.numpy as jnp
from jax import lax
from jax.experimental import pallas as pl
from jax.experimental.pallas import tpu as pltpu
from jax.experimental.pallas import tpu_sc as plsc

E = 3_200_000
N = 100_000
G = 64
H = 128
L = 32
NC, NS = 2, 16
NW = NC * NS
EPW = E // NW
CH = 10_000
NCH = EPW // CH
K = 25
NPW = 3_200
NPAD = NW * NPW - N
GP = 80

_mesh = plsc.VectorSubcoreMesh(core_axis_name="c", subcore_axis_name="s")


@functools.partial(
    pl.kernel,
    out_type=(
        tuple(jax.ShapeDtypeStruct((NW, 16 * G), jnp.float32)
              for _ in range(5))
        + (jax.ShapeDtypeStruct((NW, 16 * GP), jnp.float32),)
    ),
    mesh=_mesh,
    scratch_types=(
        [pltpu.VMEM((CH,), jnp.float32) for _ in range(4)]
        + [pltpu.VMEM((CH,), jnp.int32) for _ in range(2)]
        + [pltpu.VMEM((NPW,), jnp.int32)]
        + [pltpu.VMEM((16 * G,), jnp.float32) for _ in range(5)]
        + [pltpu.VMEM((16 * GP,), jnp.float32)]
        + [pltpu.SemaphoreType.DMA for _ in range(7)]
    ),
    compiler_params=pltpu.CompilerParams(needs_layout_passes=False),
)
def _sc_segment(eb_hbm, exa_hbm, exo_hbm, nb_hbm,
                o_sum, o_cnt, o_knw, o_max, o_evt, nt_out,
                exa0, exa1, exo0, exo1, ebb0, ebb1, nbb,
                tsum, tcnt, tknw, tmax, tevt, accn,
                sa0, sa1, so0, so1, sb0, sb1, sn):
    wid = lax.axis_index("s") * NC + lax.axis_index("c")
    lane = lax.iota(jnp.int32, 16)
    lane_g = lane * G
    lane_gp = lane * GP
    zeros = jnp.zeros((16,), jnp.float32)
    ones = jnp.ones((16,), jnp.float32)

    tabs = (tsum, tcnt, tknw, tmax, tevt)

    def zrow(r, carry):
        for t in tabs:
            t[pl.ds(r * 16, 16)] = zeros
        return carry
    lax.fori_loop(0, G, zrow, 0)

    def zrow_n(r, carry):
        accn[pl.ds(r * 16, 16)] = zeros
        return carry
    lax.fori_loop(0, GP, zrow_n, 0)

    ebase = wid * EPW
    exa = (exa0, exa1)
    exo = (exo0, exo1)
    ebb = (ebb0, ebb1)
    sa = (sa0, sa1)
    so = (so0, so1)
    sb = (sb0, sb1)

    def start_chunk(j, slot):
        sl = pl.ds(ebase + j * CH, CH)
        return (pltpu.async_copy(exa_hbm.at[sl], exa[slot], sa[slot]),
                pltpu.async_copy(exo_hbm.at[sl], exo[slot], so[slot]),
                pltpu.async_copy(eb_hbm.at[sl], ebb[slot], sb[slot]))

    pend = start_chunk(0, 0)

    pltpu.async_copy(nb_hbm.at[pl.ds(wid * NPW, NPW)], nbb, sn).wait()

    def nstep(i, carry):
        b = nbb[pl.ds(i * 16, 16)]
        plsc.addupdate_scatter(accn, [lane_gp + b], ones)
        return carry
    lax.fori_loop(0, NPW // 16, nstep, 0)

    def flush(bprev, rs, rc, rk, rm, re):
        bi = lane_g + bprev
        plsc.addupdate_scatter(tsum, [bi], rs)
        plsc.addupdate_scatter(tcnt, [bi], rc)
        plsc.addupdate_scatter(tknw, [bi], rk)
        m = plsc.load_gather(tmax, [bi])
        plsc.store_scatter(tmax, [bi], jnp.maximum(m, rm))
        e_ = plsc.load_gather(tevt, [bi])
        plsc.store_scatter(tevt, [bi], jnp.maximum(e_, re))

    kf = jnp.float32(K)
    carry0 = (jnp.zeros((16,), jnp.int32),) + (zeros,) * 10

    carry = carry0
    for j in range(NCH):
        slot = j & 1
        if j + 1 < NCH:
            nxt = start_chunk(j + 1, 1 - slot)
        for cp in pend:
            cp.wait()
        xa, xo, eb = exa[slot], exo[slot], ebb[slot]

        def block(i, c):
            bprev0 = c[0]
            b_first = eb[pl.ds(i * (16 * K), 16)]
            b_last = eb[pl.ds(i * (16 * K) + 16 * (K - 1), 16)]
            uni = jnp.logical_and(jnp.all(b_first == bprev0),
                                  jnp.all(b_last == bprev0))

            def fast(c):
                bp, rs0, rs1, rc0, rc1, rk0, rk1, rm0, rm1, re0, re1 = c
                r = [rs0, rs1, rk0, rk1, rm0, rm1, re0, re1]
                for u in range(K):
                    sl = pl.ds((i * K + u) * 16, 16)
                    a = xa[sl]
                    o = xo[sl]
                    kn = o >= 0.0
                    p = u & 1
                    r[0 + p] = r[0 + p] + a
                    r[2 + p] = r[2 + p] + jnp.where(kn, ones, zeros)
                    r[4 + p] = jnp.maximum(r[4 + p], a)
                    r[6 + p] = jnp.maximum(r[6 + p],
                                           jnp.where(kn, o + 1.0, zeros))
                return (bp, r[0], r[1], rc0 + kf, rc1,
                        r[2], r[3], r[4], r[5], r[6], r[7])

            def slow(c):
                bprev, rs0, rs1, rc0, rc1, rk0, rk1, rm0, rm1, re0, re1 = c
                rs, rc = rs0 + rs1, rc0 + rc1
                rk = rk0 + rk1
                rm, re = jnp.maximum(rm0, rm1), jnp.maximum(re0, re1)
                for u in range(K):
                    sl = pl.ds((i * K + u) * 16, 16)
                    b = eb[sl]
                    a = xa[sl]
                    o = xo[sl]
                    chg = b != bprev
                    bi = lane_g + bprev
                    plsc.addupdate_scatter(tsum, [bi], rs, mask=chg)
                    plsc.addupdate_scatter(tcnt, [bi], rc, mask=chg)
                    plsc.addupdate_scatter(tknw, [bi], rk, mask=chg)
                    m = plsc.load_gather(tmax, [bi])
                    plsc.store_scatter(tmax, [bi], jnp.maximum(m, rm),
                                       mask=chg)
                    e_ = plsc.load_gather(tevt, [bi])
                    plsc.store_scatter(tevt, [bi], jnp.maximum(e_, re),
                                       mask=chg)
                    gone = jnp.where(chg, zeros, ones)
                    kn = o >= 0.0
                    rs = rs * gone + a
                    rc = rc * gone + ones
                    rk = rk * gone + jnp.where(kn, ones, zeros)
                    rm = jnp.maximum(rm * gone, a)
                    re = jnp.maximum(re * gone,
                                     jnp.where(kn, o + 1.0, zeros))
                    bprev = b
                return (bprev, rs, zeros, rc, zeros,
                        rk, zeros, rm, zeros, re, zeros)

            return lax.cond(uni, fast, slow, c)

        carry = lax.fori_loop(0, CH // (16 * K), block, carry)
        if j + 1 < NCH:
            pend = nxt

    bp, rs0, rs1, rc0, rc1, rk0, rk1, rm0, rm1, re0, re1 = carry
    flush(bp, rs0 + rs1, rc0 + rc1, rk0 + rk1,
          jnp.maximum(rm0, rm1), jnp.maximum(re0, re1))

    outs = (o_sum, o_cnt, o_knw, o_max, o_evt)
    for t, ot in zip(tabs, outs):
        pltpu.sync_copy(t, ot.at[wid])
    pltpu.sync_copy(accn, nt_out.at[wid])


def _finish_body(ts_ref, tc_ref, tk_ref, tm_ref, te_ref, nt_ref,
                 w1t_ref, b1_ref, w2t_ref, b2_ref, out_ref):
    def rsum(ref, stride):
        x = ref[...]
        acc = x[:, 0:G]
        for l in range(1, 16):
            acc = acc + x[:, l * stride:l * stride + G]
        return acc.sum(axis=0)

    def rmax(ref, stride):
        x = ref[...]
        acc = x[:, 0:G]
        for l in range(1, 16):
            acc = jnp.maximum(acc, x[:, l * stride:l * stride + G])
        return acc.max(axis=0)

    nn = rsum(nt_ref, GP)
    s_ = rsum(ts_ref, G)
    c_ = rsum(tc_ref, G)
    k_ = rsum(tk_ref, G)
    m_ = rmax(tm_ref, G)
    ev = rmax(te_ref, G)
    denom = jnp.maximum(c_, 1.0)
    feats_t = jnp.concatenate(
        [nn[None], c_[None], (s_ / denom)[None], m_[None],
         (k_ / denom)[None], ev[None]], axis=0)
    h_t = jnp.maximum(
        jnp.dot(w1t_ref[...], feats_t,
                preferred_element_type=jnp.float32) + b1_ref[...], 0.0)
    out = lax.dot_general(
        h_t, w2t_ref[...], (((0,), (1,)), ((), ())),
        preferred_element_type=jnp.float32)
    out_ref[...] = out + b2_ref[...]


def kernel(node_x, edge_x, node_batch, edge_batch, W1, b1, W2, b2):
    arit = edge_x[:, :1].reshape(E)
    orig = edge_x[:, 1:].reshape(E)
    nb = jnp.concatenate(
        [node_batch, jnp.full((NPAD,), G, jnp.int32)])
    *tables, nt = _sc_segment(edge_batch, arit, orig, nb)
    return pl.pallas_call(
        _finish_body,
        out_shape=jax.ShapeDtypeStruct((G, L), jnp.float32),
    )(*tables, nt, W1.T, b1.reshape(H, 1), W2.T, b2.reshape(1, L))
